# Initial kernel scaffold; baseline (speedup 1.0000x reference)
import jax
import jax.numpy as jnp
from jax.experimental import pallas as pl


def kernel(h, r, t, entity_embeddings, relation_embeddings):
    raise NotImplementedError("write your pallas kernel here")


# speedup vs baseline: 1.1236x; 1.1236x over previous
import functools

import jax
import jax.numpy as jnp
from jax import lax
from jax.experimental import pallas as pl
from jax.experimental.pallas import tpu as pltpu
from jax.experimental.pallas import tpu_sc as plsc

NUM_ENTITIES = 100000
NUM_RELATIONS = 1000
D = 128
B = 16384
L = 16

NC = 2
NS = 16
NW = NC * NS
RPW = B // NW
CHUNK = 128
NCHUNK = RPW // CHUNK


def _body(h_hbm, r_hbm, t_hbm, ent_hbm, rel_hbm, out_hbm,
          ih, ir, it, hrow, rrow, trow, outbuf, sem):
    wid = lax.axis_index("s") * NC + lax.axis_index("c")
    base = wid * RPW

    lane = lax.iota(jnp.int32, L)

    for c in range(NCHUNK):
        off = c * CHUNK
        pltpu.sync_copy(h_hbm.at[pl.ds(base + off, CHUNK)], ih)
        pltpu.sync_copy(r_hbm.at[pl.ds(base + off, CHUNK)], ir)
        pltpu.sync_copy(t_hbm.at[pl.ds(base + off, CHUNK)], it)
        cp_h = pltpu.make_async_copy(ent_hbm.at[ih], hrow, sem)
        cp_r = pltpu.make_async_copy(rel_hbm.at[ir], rrow, sem)
        cp_t = pltpu.make_async_copy(ent_hbm.at[it], trow, sem)
        cp_h.start()
        cp_r.start()
        cp_t.start()
        cp_h.wait()
        cp_r.wait()
        cp_t.wait()

        def group_body(g, _, off=off):
            rowbase = g * L
            sv = jnp.zeros((L,), jnp.float32)
            for j in range(L):
                row = rowbase + j
                p = (hrow[row, pl.ds(0, L)]
                     * rrow[row, pl.ds(0, L)]
                     * trow[row, pl.ds(0, L)])
                for k in range(1, D // L):
                    p = p + (hrow[row, pl.ds(k * L, L)]
                             * rrow[row, pl.ds(k * L, L)]
                             * trow[row, pl.ds(k * L, L)])
                s = jnp.sum(p)
                sv = jnp.where(lane == j, s, sv)
            outbuf[pl.ds(off + rowbase, L)] = sv
            return 0

        lax.fori_loop(0, CHUNK // L, group_body, 0)

    pltpu.sync_copy(outbuf, out_hbm.at[pl.ds(base, RPW)])


def _distmult_sc(h, r, t, ent, rel):
    mesh = plsc.VectorSubcoreMesh(core_axis_name="c", subcore_axis_name="s")
    k = functools.partial(
        pl.kernel,
        out_type=jax.ShapeDtypeStruct((B,), jnp.float32),
        mesh=mesh,
        compiler_params=pltpu.CompilerParams(needs_layout_passes=False),
        scratch_types=[
            pltpu.VMEM((CHUNK,), jnp.int32),
            pltpu.VMEM((CHUNK,), jnp.int32),
            pltpu.VMEM((CHUNK,), jnp.int32),
            pltpu.VMEM((CHUNK, D), jnp.float32),
            pltpu.VMEM((CHUNK, D), jnp.float32),
            pltpu.VMEM((CHUNK, D), jnp.float32),
            pltpu.VMEM((RPW,), jnp.float32),
            pltpu.SemaphoreType.DMA,
        ],
    )(_body)
    return k(h, r, t, ent, rel)


def kernel(h, r, t, entity_embeddings, relation_embeddings):
    h = jnp.asarray(h, jnp.int32)
    r = jnp.asarray(r, jnp.int32)
    t = jnp.asarray(t, jnp.int32)
    return _distmult_sc(h, r, t, entity_embeddings, relation_embeddings)


# speedup vs baseline: 2.1846x; 1.9443x over previous
import functools

import jax
import jax.numpy as jnp
from jax import lax
from jax.experimental import pallas as pl
from jax.experimental.pallas import tpu as pltpu
from jax.experimental.pallas import tpu_sc as plsc

NUM_ENTITIES = 100000
NUM_RELATIONS = 1000
D = 128
B = 16384
L = 16

NC = 2
NS = 16
NW = NC * NS
RPW = B // NW
CHUNK = 128
NCHUNK = RPW // CHUNK


def _body(h_hbm, r_hbm, t_hbm, ent_hbm, rel_hbm, out_hbm,
          ihall, irall, itall,
          hrow0, rrow0, trow0, hrow1, rrow1, trow1,
          pacc, outbuf, sem0, sem1):
    wid = lax.axis_index("s") * NC + lax.axis_index("c")
    base = wid * RPW
    lane = lax.iota(jnp.int32, L)
    colbase = lane * L

    pltpu.sync_copy(h_hbm.at[pl.ds(base, RPW)], ihall)
    pltpu.sync_copy(r_hbm.at[pl.ds(base, RPW)], irall)
    pltpu.sync_copy(t_hbm.at[pl.ds(base, RPW)], itall)

    bufs = [(hrow0, rrow0, trow0), (hrow1, rrow1, trow1)]
    sems = [sem0, sem1]

    def fire(c):
        hb, rb, tb = bufs[c % 2]
        s = sems[c % 2]
        sl = pl.ds(c * CHUNK, CHUNK)
        cps = (pltpu.make_async_copy(ent_hbm.at[ihall.at[sl]], hb, s),
               pltpu.make_async_copy(rel_hbm.at[irall.at[sl]], rb, s),
               pltpu.make_async_copy(ent_hbm.at[itall.at[sl]], tb, s))
        for cp in cps:
            cp.start()
        return cps

    def compute(c):
        hb, rb, tb = bufs[c % 2]
        off = c * CHUNK

        def group_body(g, _):
            rowbase = g * L
            for j in range(L):
                row = rowbase + j
                p = (hb[row, pl.ds(0, L)]
                     * rb[row, pl.ds(0, L)]
                     * tb[row, pl.ds(0, L)])
                for k in range(1, D // L):
                    p = p + (hb[row, pl.ds(k * L, L)]
                             * rb[row, pl.ds(k * L, L)]
                             * tb[row, pl.ds(k * L, L)])
                pacc[pl.ds(j * L, L)] = p
            s = plsc.load_gather(pacc, [colbase])
            for l in range(1, L):
                s = s + plsc.load_gather(pacc, [colbase + l])
            outbuf[pl.ds(off + rowbase, L)] = s
            return 0

        lax.fori_loop(0, CHUNK // L, group_body, 0)

    pending = fire(0)
    for c in range(NCHUNK):
        nxt = fire(c + 1) if c + 1 < NCHUNK else None
        for cp in pending:
            cp.wait()
        compute(c)
        pending = nxt

    pltpu.sync_copy(outbuf, out_hbm.at[pl.ds(base, RPW)])


def _distmult_sc(h, r, t, ent, rel):
    mesh = plsc.VectorSubcoreMesh(core_axis_name="c", subcore_axis_name="s")
    k = functools.partial(
        pl.kernel,
        out_type=jax.ShapeDtypeStruct((B,), jnp.float32),
        mesh=mesh,
        compiler_params=pltpu.CompilerParams(needs_layout_passes=False),
        scratch_types=[
            pltpu.VMEM((RPW,), jnp.int32),
            pltpu.VMEM((RPW,), jnp.int32),
            pltpu.VMEM((RPW,), jnp.int32),
            pltpu.VMEM((CHUNK, D), jnp.float32),
            pltpu.VMEM((CHUNK, D), jnp.float32),
            pltpu.VMEM((CHUNK, D), jnp.float32),
            pltpu.VMEM((CHUNK, D), jnp.float32),
            pltpu.VMEM((CHUNK, D), jnp.float32),
            pltpu.VMEM((CHUNK, D), jnp.float32),
            pltpu.VMEM((L * L,), jnp.float32),
            pltpu.VMEM((RPW,), jnp.float32),
            pltpu.SemaphoreType.DMA,
            pltpu.SemaphoreType.DMA,
        ],
    )(_body)
    return k(h, r, t, ent, rel)


def kernel(h, r, t, entity_embeddings, relation_embeddings):
    h = jnp.asarray(h, jnp.int32)
    r = jnp.asarray(r, jnp.int32)
    t = jnp.asarray(t, jnp.int32)
    return _distmult_sc(h, r, t, entity_embeddings, relation_embeddings)
